# baseline (speedup 1.0000x reference)
import functools

import jax
import jax.numpy as jnp
from jax.experimental import pallas as pl
from jax.experimental.pallas import tpu as pltpu

_INTERPRET = False


def _cooc_kernel(nodes_ref, nodesT_ref, mask_ref, maskT_ref, ker_ref, out_ref):
    nodes = nodes_ref[0]
    nodesT = nodesT_ref[0]
    mask = mask_ref[0]
    maskT = maskT_ref[0]
    ker = ker_ref[...]
    W, L = nodes.shape

    wi = jax.lax.broadcasted_iota(jnp.int32, (W, L), 0)
    pi = jax.lax.broadcasted_iota(jnp.int32, (W, L), 1)
    nm = jnp.where(mask != 0.0, nodes, -1 - (wi * L + pi))
    wiT = jax.lax.broadcasted_iota(jnp.int32, (L, W), 1)
    piT = jax.lax.broadcasted_iota(jnp.int32, (L, W), 0)
    nmT = jnp.where(maskT != 0.0, nodesT, -1 - (wiT * L + piT))

    kq = jnp.round(ker * 65536.0).astype(jnp.int32) + (1 << 21)
    acc_i = jnp.zeros((W, W), jnp.int32)
    corr = jnp.zeros((W, 1), jnp.float32)
    for p1 in range(L):
        n1 = jax.lax.slice(nm, (0, p1), (W, p1 + 1))
        packed = jnp.zeros((W, W), jnp.int32)
        for p2 in range(L):
            n2 = jax.lax.slice(nmT, (p2, 0), (p2 + 1, W))
            eq = n1 == n2
            packed = packed + jnp.where(eq, kq[p1, p2], 0)
        acc_i = acc_i + (packed & ((1 << 21) - 1))
        esum = packed >> 21
        cnt = jnp.sum(esum, axis=1, keepdims=True)
        corr = corr + jnp.where(cnt == 1, ker[p1, p1], 0.0)
    acc = acc_i.astype(jnp.float32) * (1.0 / 65536.0)

    ri = jax.lax.broadcasted_iota(jnp.int32, (W, W), 0)
    ci = jax.lax.broadcasted_iota(jnp.int32, (W, W), 1)
    acc = acc - jnp.where(ri == ci, corr, 0.0)

    lens_c = jnp.sum(mask, axis=1, keepdims=True)
    lens_r = jnp.sum(maskT, axis=0, keepdims=True)
    norm = jnp.maximum(lens_c * lens_r, 1e-6)
    valid = (lens_c > 0.0) & (lens_r > 0.0)
    co = jnp.where(valid, acc / norm, 0.0)
    co = jnp.clip(co, -10.0, 10.0)
    out_ref[0] = jnp.tanh(co)


def kernel(anonymized_nodes, walk_masks, kernel):
    B, W, L = anonymized_nodes.shape
    nodesT = jnp.swapaxes(anonymized_nodes, 1, 2)
    maskT = jnp.swapaxes(walk_masks, 1, 2)
    out = pl.pallas_call(
        _cooc_kernel,
        grid=(B,),
        in_specs=[
            pl.BlockSpec((1, W, L), lambda b: (b, 0, 0)),
            pl.BlockSpec((1, L, W), lambda b: (b, 0, 0)),
            pl.BlockSpec((1, W, L), lambda b: (b, 0, 0)),
            pl.BlockSpec((1, L, W), lambda b: (b, 0, 0)),
            pl.BlockSpec((L, L), lambda b: (0, 0)),
        ],
        out_specs=pl.BlockSpec((1, W, W), lambda b: (b, 0, 0)),
        out_shape=jax.ShapeDtypeStruct((B, W, W), jnp.float32),
        compiler_params=pltpu.CompilerParams(
            dimension_semantics=("arbitrary",),
        ),
        interpret=_INTERPRET,
    )(anonymized_nodes, nodesT, walk_masks, maskT, kernel)
    return out


# speedup vs baseline: 2.2493x; 2.2493x over previous
import functools

import jax
import jax.numpy as jnp
from jax import lax
from jax.experimental import pallas as pl
from jax.experimental.pallas import tpu as pltpu
from jax.experimental.pallas import tpu_sc as plsc

_NLANE = 16


def _sc_body(nodes_hbm, mask_hbm, ker_hbm, enc_hbm, out_hbm,
             nodes_v, mask_v, ker_v, enc_v, histT, hist, offs, pcur,
             senc, snode, co, lens, invl, W, L, NC):
    N = W * L
    V = 1024
    NCH = N // _NLANE
    CELLS = W * W

    wid = lax.axis_index("s") * NC + lax.axis_index("c")
    pltpu.sync_copy(nodes_hbm.at[wid], nodes_v)
    pltpu.sync_copy(mask_hbm.at[wid], mask_v)
    pltpu.sync_copy(ker_hbm, ker_v)
    pltpu.sync_copy(enc_hbm, enc_v)

    lane = lax.iota(jnp.int32, _NLANE)
    zi = jnp.zeros((_NLANE,), jnp.int32)
    zf = jnp.zeros((_NLANE,), jnp.float32)

    def zero_body(i, _):
        co[pl.ds(i * _NLANE, _NLANE)] = zf
        histT[pl.ds(i * _NLANE, _NLANE)] = zi
        return 0
    lax.fori_loop(0, CELLS // _NLANE, zero_body, 0)

    def zero_small(i, _):
        lens[pl.ds(i * _NLANE, _NLANE)] = zf
        return 0
    lax.fori_loop(0, W // _NLANE, zero_small, 0)

    def hist_body(i, _):
        b = i * _NLANE
        idx = nodes_v[pl.ds(b, _NLANE)]
        mval = mask_v[pl.ds(b, _NLANE)]
        valid = mval != 0.0
        w = enc_v[pl.ds(b, _NLANE)] >> 8
        plsc.addupdate_scatter(lens, [w], mval)
        plsc.addupdate_scatter(histT, [lane * V + idx],
                               jnp.ones((_NLANE,), jnp.int32), mask=valid)
        return 0
    lax.fori_loop(0, NCH, hist_body, 0)

    def red_body(j, _):
        b = j * _NLANE
        acc = histT[pl.ds(b, _NLANE)]
        for l in range(1, _NLANE):
            acc = acc + histT[pl.ds(l * V + b, _NLANE)]
        hist[pl.ds(b, _NLANE)] = acc
        return 0
    lax.fori_loop(0, V // _NLANE, red_body, 0)

    def scan_body(j, carry):
        b = j * _NLANE
        seg = hist[pl.ds(b, _NLANE)]
        cs = plsc.cumsum(seg)
        offs[pl.ds(b, _NLANE)] = cs - seg + carry
        return carry + jnp.sum(seg)
    nvalid = lax.fori_loop(0, V // _NLANE, scan_body, jnp.int32(0))

    def cur_body(j, _):
        b = j * _NLANE
        acc = offs[pl.ds(b, _NLANE)]
        for l in range(_NLANE):
            pcur[pl.ds(l * V + b, _NLANE)] = acc
            acc = acc + histT[pl.ds(l * V + b, _NLANE)]
        return 0
    lax.fori_loop(0, V // _NLANE, cur_body, 0)

    def place_body(i, _):
        b = i * _NLANE
        idx = nodes_v[pl.ds(b, _NLANE)]
        valid = mask_v[pl.ds(b, _NLANE)] != 0.0
        ev = enc_v[pl.ds(b, _NLANE)]
        flat = lane * V + idx
        pos = plsc.load_gather(pcur, [flat], mask=valid)
        plsc.store_scatter(senc, [pos], ev, mask=valid)
        plsc.store_scatter(snode, [pos], idx, mask=valid)
        plsc.store_scatter(pcur, [flat], pos + 1, mask=valid)
        return 0
    lax.fori_loop(0, NCH, place_body, 0)

    def invl_body(i, _):
        b = i * _NLANE
        lv = lens[pl.ds(b, _NLANE)]
        invl[pl.ds(b, _NLANE)] = 1.0 / jnp.maximum(lv, 1.0)
        return 0
    lax.fori_loop(0, W // _NLANE, invl_body, 0)

    def emit_body(i, _):
        b = i * _NLANE
        svec = b + lane
        act0 = svec < nvalid
        encs = senc[pl.ds(b, _NLANE)]
        vs = snode[pl.ds(b, _NLANE)]
        a = plsc.load_gather(offs, [vs], mask=act0)
        m = plsc.load_gather(hist, [vs], mask=act0)
        m = jnp.where(act0, m, 0)
        seg_ok = act0 & (m >= 2)
        mlen = jnp.max(m)
        cell0 = (encs >> 8) * W
        p20 = (encs & 255) * L

        def cond(c):
            return c[0] < mlen

        def body(c):
            k = c[0]
            act = seg_ok & (k < m)
            je = plsc.load_gather(senc, [a + k], mask=act)
            kv = plsc.load_gather(ker_v, [p20 + (je & 255)], mask=act)
            plsc.addupdate_scatter(co, [cell0 + (je >> 8)], kv, mask=act)
            return (k + 1,)
        lax.while_loop(cond, body, (jnp.int32(0),))
        return 0
    lax.fori_loop(0, NCH, emit_body, 0)

    def ep_body(i, _):
        c0 = i * _NLANE
        x = co[pl.ds(c0, _NLANE)]
        w1 = jnp.broadcast_to((c0 >> 7).astype(jnp.int32), (_NLANE,))
        il1 = plsc.load_gather(invl, [w1])
        il2 = invl[pl.ds(c0 & (W - 1), _NLANE)]
        y = x * il1 * il2
        y = jnp.clip(y, -10.0, 10.0)
        e = jnp.exp(2.0 * y)
        co[pl.ds(c0, _NLANE)] = 1.0 - 2.0 / (e + 1.0)
        return 0
    lax.fori_loop(0, CELLS // _NLANE, ep_body, 0)
    pltpu.sync_copy(co, out_hbm.at[wid])


def kernel(anonymized_nodes, walk_masks, kernel):
    B, W, L = anonymized_nodes.shape
    N = W * L
    V = 1024
    info = plsc.get_sparse_core_info()
    NC = info.num_cores
    ker = jnp.clip(kernel[:L, :L], -10.0, 10.0).reshape(L * L)
    nodes_flat = anonymized_nodes.reshape(B, N)
    mask_flat = walk_masks.reshape(B, N)
    ar = jnp.arange(N, dtype=jnp.int32)
    enc = (ar // L) * 256 + (ar % L)

    mesh = plsc.VectorSubcoreMesh(core_axis_name="c", subcore_axis_name="s")
    body = functools.partial(_sc_body, W=W, L=L, NC=NC)
    run = pl.kernel(
        body,
        out_type=jax.ShapeDtypeStruct((B, W * W), jnp.float32),
        mesh=mesh,
        scratch_types=[
            pltpu.VMEM((N,), jnp.int32),
            pltpu.VMEM((N,), jnp.float32),
            pltpu.VMEM((L * L,), jnp.float32),
            pltpu.VMEM((N,), jnp.int32),
            pltpu.VMEM((_NLANE * V,), jnp.int32),
            pltpu.VMEM((V,), jnp.int32),
            pltpu.VMEM((V,), jnp.int32),
            pltpu.VMEM((_NLANE * V,), jnp.int32),
            pltpu.VMEM((N,), jnp.int32),
            pltpu.VMEM((N,), jnp.int32),
            pltpu.VMEM((W * W,), jnp.float32),
            pltpu.VMEM((W,), jnp.float32),
            pltpu.VMEM((W,), jnp.float32),
        ],
        compiler_params=pltpu.CompilerParams(needs_layout_passes=False),
    )
    out = run(nodes_flat, mask_flat, ker, enc)
    return out.reshape(B, W, W)


# speedup vs baseline: 2.4588x; 1.0931x over previous
import functools

import jax
import jax.numpy as jnp
from jax import lax
from jax.experimental import pallas as pl
from jax.experimental.pallas import tpu as pltpu
from jax.experimental.pallas import tpu_sc as plsc

_NLANE = 16


def _sc_body(nodes_hbm, mask_hbm, ker_hbm, enc_hbm, out_hbm,
             nodes_v, mask_v, ker_v, enc_v, histT, hist, offs, pcur,
             senc, snode, co, lens, invl, W, L, NC):
    N = W * L
    V = 1024
    NCH = N // _NLANE
    CELLS = W * W

    wid = lax.axis_index("s") * NC + lax.axis_index("c")
    pltpu.sync_copy(nodes_hbm.at[wid], nodes_v)
    pltpu.sync_copy(mask_hbm.at[wid], mask_v)
    pltpu.sync_copy(ker_hbm, ker_v)
    pltpu.sync_copy(enc_hbm, enc_v)

    lane = lax.iota(jnp.int32, _NLANE)
    zi = jnp.zeros((_NLANE,), jnp.int32)
    zf = jnp.zeros((_NLANE,), jnp.float32)

    def zero_body(i, _):
        for j in range(8):
            co[pl.ds(i * 128 + j * _NLANE, _NLANE)] = zf
            histT[pl.ds(i * 128 + j * _NLANE, _NLANE)] = zi
        return 0
    lax.fori_loop(0, CELLS // 128, zero_body, 0)

    def zero_small(i, _):
        lens[pl.ds(i * _NLANE, _NLANE)] = zf
        return 0
    lax.fori_loop(0, W // _NLANE, zero_small, 0)

    ones_i = jnp.ones((_NLANE,), jnp.int32)

    def hist_body(i, _):
        for j in range(4):
            b = i * 4 * _NLANE + j * _NLANE
            idx = nodes_v[pl.ds(b, _NLANE)]
            mval = mask_v[pl.ds(b, _NLANE)]
            valid = mval != 0.0
            w = enc_v[pl.ds(b, _NLANE)] >> 8
            plsc.addupdate_scatter(lens, [w], mval)
            plsc.addupdate_scatter(histT, [lane * V + idx], ones_i, mask=valid)
        return 0
    lax.fori_loop(0, NCH // 4, hist_body, 0)

    def seg_body(j, carry):
        b = j * _NLANE
        acc = histT[pl.ds(b, _NLANE)]
        for l in range(1, _NLANE):
            acc = acc + histT[pl.ds(l * V + b, _NLANE)]
        hist[pl.ds(b, _NLANE)] = acc
        cs = plsc.cumsum(acc)
        off = cs - acc + carry
        offs[pl.ds(b, _NLANE)] = off
        for l in range(_NLANE):
            pcur[pl.ds(l * V + b, _NLANE)] = off
            off = off + histT[pl.ds(l * V + b, _NLANE)]
        return carry + jnp.sum(acc)
    nvalid = lax.fori_loop(0, V // _NLANE, seg_body, jnp.int32(0))

    def place_body(i, _):
        for j in range(4):
            b = i * 4 * _NLANE + j * _NLANE
            idx = nodes_v[pl.ds(b, _NLANE)]
            valid = mask_v[pl.ds(b, _NLANE)] != 0.0
            ev = enc_v[pl.ds(b, _NLANE)]
            flat = lane * V + idx
            pos = plsc.load_gather(pcur, [flat], mask=valid)
            plsc.store_scatter(senc, [pos], ev, mask=valid)
            plsc.store_scatter(snode, [pos], idx, mask=valid)
            plsc.store_scatter(pcur, [flat], pos + 1, mask=valid)
        return 0
    lax.fori_loop(0, NCH // 4, place_body, 0)

    def invl_body(i, _):
        b = i * _NLANE
        lv = lens[pl.ds(b, _NLANE)]
        invl[pl.ds(b, _NLANE)] = 1.0 / jnp.maximum(lv, 1.0)
        return 0
    lax.fori_loop(0, W // _NLANE, invl_body, 0)

    def emit_body(i, _):
        b = i * _NLANE
        svec = b + lane
        act0 = svec < nvalid
        encs = senc[pl.ds(b, _NLANE)]
        vs = snode[pl.ds(b, _NLANE)]
        a = plsc.load_gather(offs, [vs], mask=act0)
        m = plsc.load_gather(hist, [vs], mask=act0)
        m = jnp.where(act0, m, 0)
        seg_ok = act0 & (m >= 2)
        mlen = jnp.max(m)
        cell0 = (encs >> 8) * W
        p20 = (encs & 255) * L

        def cond(c):
            return c[0] < mlen

        def body(c):
            k = c[0]
            for u in range(2):
                ku = k + u
                act = seg_ok & (ku < m)
                je = plsc.load_gather(senc, [a + ku], mask=act)
                kv = plsc.load_gather(ker_v, [p20 + (je & 255)], mask=act)
                plsc.addupdate_scatter(co, [cell0 + (je >> 8)], kv, mask=act)
            return (k + 2,)
        lax.while_loop(cond, body, (jnp.int32(0),))
        return 0
    lax.fori_loop(0, NCH, emit_body, 0)

    def ep_body(i, _):
        w1 = jnp.broadcast_to(i.astype(jnp.int32), (_NLANE,))
        il1 = plsc.load_gather(invl, [w1])
        for j in range(W // _NLANE):
            c0 = i * W + j * _NLANE
            x = co[pl.ds(c0, _NLANE)]
            il2 = invl[pl.ds(j * _NLANE, _NLANE)]
            y = x * il1 * il2
            y = jnp.clip(y, -10.0, 10.0)
            e = jnp.exp(2.0 * y)
            co[pl.ds(c0, _NLANE)] = 1.0 - 2.0 / (e + 1.0)
        return 0
    lax.fori_loop(0, W, ep_body, 0)
    pltpu.sync_copy(co, out_hbm.at[wid])


def kernel(anonymized_nodes, walk_masks, kernel):
    B, W, L = anonymized_nodes.shape
    N = W * L
    V = 1024
    info = plsc.get_sparse_core_info()
    NC = info.num_cores
    ker = jnp.clip(kernel[:L, :L], -10.0, 10.0).reshape(L * L)
    nodes_flat = anonymized_nodes.reshape(B, N)
    mask_flat = walk_masks.reshape(B, N)
    ar = jnp.arange(N, dtype=jnp.int32)
    enc = (ar // L) * 256 + (ar % L)

    mesh = plsc.VectorSubcoreMesh(core_axis_name="c", subcore_axis_name="s")
    body = functools.partial(_sc_body, W=W, L=L, NC=NC)
    run = pl.kernel(
        body,
        out_type=jax.ShapeDtypeStruct((B, W * W), jnp.float32),
        mesh=mesh,
        scratch_types=[
            pltpu.VMEM((N,), jnp.int32),
            pltpu.VMEM((N,), jnp.float32),
            pltpu.VMEM((L * L,), jnp.float32),
            pltpu.VMEM((N,), jnp.int32),
            pltpu.VMEM((_NLANE * V,), jnp.int32),
            pltpu.VMEM((V,), jnp.int32),
            pltpu.VMEM((V,), jnp.int32),
            pltpu.VMEM((_NLANE * V,), jnp.int32),
            pltpu.VMEM((N,), jnp.int32),
            pltpu.VMEM((N,), jnp.int32),
            pltpu.VMEM((W * W,), jnp.float32),
            pltpu.VMEM((W,), jnp.float32),
            pltpu.VMEM((W,), jnp.float32),
        ],
        compiler_params=pltpu.CompilerParams(needs_layout_passes=False),
    )
    out = run(nodes_flat, mask_flat, ker, enc)
    return out.reshape(B, W, W)
